# Initial kernel scaffold; baseline (speedup 1.0000x reference)
import jax
import jax.numpy as jnp
from jax.experimental import pallas as pl


def kernel(x, edge_index, W1l, b1, W1r, W2l, b2, W2r):
    raise NotImplementedError("write your pallas kernel here")


# speedup vs baseline: 6.6027x; 6.6027x over previous
import functools

import jax
import jax.numpy as jnp
from jax import lax
from jax.experimental import pallas as pl
from jax.experimental.pallas import tpu as pltpu
from jax.experimental.pallas import tpu_sc as plsc

N_CORES = 2
N_SUBCORES = 16
STREAM = 128


def _make_seg_sum(n_nodes, n_streams_total, with_cnt):
    own = -(-(n_nodes + 8) // (N_SUBCORES * STREAM)) * STREAM
    n_rows_acc = own * N_SUBCORES
    n_chunks = own // STREAM
    last0 = (N_SUBCORES - 1) * own
    full_last = (n_nodes - last0) // STREAM
    tail = n_nodes - last0 - full_last * STREAM
    streams_per_tile = n_streams_total // N_SUBCORES

    out_type = [
        jax.ShapeDtypeStruct((n_nodes, 128), jnp.float32),
        jax.ShapeDtypeStruct((n_nodes, 128), jnp.float32),
    ]
    scratch = [
        pltpu.VMEM((STREAM,), jnp.int32),
        pltpu.VMEM((STREAM,), jnp.int32),
        pltpu.VMEM((STREAM, 128), jnp.float32),
        pltpu.VMEM_SHARED((n_rows_acc, 128), jnp.float32),
        pltpu.SemaphoreType.DMA,
    ]
    if with_cnt:
        out_type.append(jax.ShapeDtypeStruct((n_nodes, 16), jnp.float32))
        scratch.append(pltpu.VMEM((STREAM, 16), jnp.float32))
        scratch.append(pltpu.VMEM_SHARED((n_rows_acc, 16), jnp.float32))

    mesh = plsc.VectorSubcoreMesh(core_axis_name="c", subcore_axis_name="s")

    def body(ylo, yhi, srcp, dstp, zfeat, zcnt, ones_h, *rest):
        if with_cnt:
            (out_lo, out_hi, out_cnt, isrc, idst, rows, acc, sem,
             onesv, cacc) = rest
        else:
            out_lo, out_hi, isrc, idst, rows, acc, sem = rest
        c = lax.axis_index("c")
        s = lax.axis_index("s")
        r0 = s * own

        pltpu.sync_copy(zfeat, rows)
        for m in range(n_chunks):
            pltpu.sync_copy(rows,
                            acc.at[pl.ds(r0 + m * STREAM, STREAM)])

        if with_cnt:
            @pl.when(c == 1)
            def _():
                pltpu.sync_copy(zcnt, onesv)
                for m in range(n_chunks):
                    pltpu.sync_copy(onesv,
                                    cacc.at[pl.ds(r0 + m * STREAM, STREAM)])
                pltpu.sync_copy(ones_h, onesv)

        plsc.subcore_barrier()

        g0 = s * streams_per_tile

        def iter_body(k, carry):
            off = (g0 + k) * STREAM
            pltpu.sync_copy(srcp.at[pl.ds(off, STREAM)], isrc)
            pltpu.sync_copy(dstp.at[pl.ds(off, STREAM)], idst)

            @pl.when(c == 0)
            def _():
                pltpu.async_copy(ylo.at[isrc], rows, sem).wait()

            @pl.when(c == 1)
            def _():
                pltpu.async_copy(yhi.at[isrc], rows, sem).wait()

            pltpu.sync_copy(rows, acc.at[idst], add=True)
            if with_cnt:
                @pl.when(c == 1)
                def _():
                    pltpu.sync_copy(onesv, cacc.at[idst], add=True)
            return carry

        lax.fori_loop(0, streams_per_tile, iter_body, 0)

        plsc.subcore_barrier()

        out_feat = [out_lo, out_hi]

        def wb_feat(base, nrows):
            rst = rows.at[pl.ds(0, nrows)]
            pltpu.sync_copy(acc.at[pl.ds(base, nrows)], rst)
            for cc in range(N_CORES):
                @pl.when(c == cc)
                def _():
                    pltpu.sync_copy(rst, out_feat[cc].at[pl.ds(base, nrows)])

        def wb_cnt(base, nrows):
            cst = onesv.at[pl.ds(0, nrows)]

            @pl.when(c == 1)
            def _():
                pltpu.sync_copy(cacc.at[pl.ds(base, nrows)], cst)
                pltpu.sync_copy(cst, out_cnt.at[pl.ds(base, nrows)])

        @pl.when(s < N_SUBCORES - 1)
        def _():
            for m in range(n_chunks):
                wb_feat(r0 + m * STREAM, STREAM)
                if with_cnt:
                    wb_cnt(r0 + m * STREAM, STREAM)

        @pl.when(s == N_SUBCORES - 1)
        def _():
            for m in range(full_last):
                wb_feat(last0 + m * STREAM, STREAM)
                if with_cnt:
                    wb_cnt(last0 + m * STREAM, STREAM)
            if tail:
                wb_feat(last0 + full_last * STREAM, tail)
                if with_cnt:
                    wb_cnt(last0 + full_last * STREAM, tail)

    return pl.kernel(
        body, out_type=out_type, mesh=mesh, scratch_types=scratch,
        compiler_params=pltpu.CompilerParams(use_tc_tiling_on_sc=False),
        name="seg_sum_sc")


def _tc1_body(x_ref, wl_ref, wr_ref, ylo_ref, yhi_ref, z_ref):
    xb = x_ref[...]
    yl = jnp.dot(xb, wl_ref[...], preferred_element_type=jnp.float32)
    ylo_ref[...] = yl[:, :128]
    yhi_ref[...] = yl[:, 128:]
    z_ref[...] = jnp.dot(xb, wr_ref[...], preferred_element_type=jnp.float32)


def _tc2_body(slo_ref, shi_ref, cnt_ref, z1_ref, b1_ref, w2l_ref, w2r_ref,
              ylo_ref, yhi_ref, z2_ref):
    inv = 1.0 / jnp.maximum(cnt_ref[...][:, 0:1], 1.0)
    sfull = jnp.concatenate([slo_ref[...], shi_ref[...]], axis=1)
    h = jnp.maximum(sfull * inv + b1_ref[...] + z1_ref[...], 0.0)
    y2 = jnp.dot(h, w2l_ref[...], preferred_element_type=jnp.float32)
    ylo_ref[...] = y2[:, :128]
    yhi_ref[...] = y2[:, 128:]
    z2_ref[...] = jnp.dot(h, w2r_ref[...], preferred_element_type=jnp.float32)


def _tc3_body(slo_ref, shi_ref, cnt_ref, z2_ref, b2_ref, out_ref):
    inv = 1.0 / jnp.maximum(cnt_ref[...][:, 0:1], 1.0)
    sfull = jnp.concatenate([slo_ref[...], shi_ref[...]], axis=1)
    out_ref[...] = sfull * inv + b2_ref[...] + z2_ref[...]


def _row_spec(nb, w):
    return pl.BlockSpec((nb, w), lambda i: (i, 0))


def _full_spec(shape):
    return pl.BlockSpec(shape, lambda i: tuple(0 for _ in shape))


def kernel(x, edge_index, W1l, b1, W1r, W2l, b2, W2r):
    n, d_in = x.shape
    h_dim = W1l.shape[1]
    e = edge_index.shape[1]

    chunk = N_SUBCORES * STREAM
    e_pad = ((e + chunk - 1) // chunk) * chunk
    n_streams = e_pad // STREAM
    pad = e_pad - e
    src_p = jnp.concatenate([edge_index[0], jnp.zeros((pad,), jnp.int32)])
    dst_p = jnp.concatenate([edge_index[1], jnp.full((pad,), n, jnp.int32)])

    zfeat = jnp.zeros((STREAM, 128), jnp.float32)
    zcnt = jnp.zeros((STREAM, 16), jnp.float32)
    ones_h = jnp.ones((STREAM, 16), jnp.float32)

    nb = 1000
    grid = (n // nb,)

    y1lo, y1hi, z1 = pl.pallas_call(
        _tc1_body,
        grid=grid,
        in_specs=[_row_spec(nb, d_in), _full_spec((d_in, h_dim)),
                  _full_spec((d_in, h_dim))],
        out_specs=[_row_spec(nb, 128), _row_spec(nb, 128),
                   _row_spec(nb, h_dim)],
        out_shape=[jax.ShapeDtypeStruct((n, 128), jnp.float32),
                   jax.ShapeDtypeStruct((n, 128), jnp.float32),
                   jax.ShapeDtypeStruct((n, h_dim), jnp.float32)],
    )(x, W1l, W1r)

    seg1 = _make_seg_sum(n, n_streams, with_cnt=True)
    s1lo, s1hi, cnt16 = seg1(y1lo, y1hi, src_p, dst_p, zfeat, zcnt, ones_h)

    y2lo, y2hi, z2 = pl.pallas_call(
        _tc2_body,
        grid=grid,
        in_specs=[_row_spec(nb, 128), _row_spec(nb, 128), _row_spec(nb, 16),
                  _row_spec(nb, h_dim), _full_spec((1, h_dim)),
                  _full_spec((h_dim, h_dim)), _full_spec((h_dim, h_dim))],
        out_specs=[_row_spec(nb, 128), _row_spec(nb, 128),
                   _row_spec(nb, h_dim)],
        out_shape=[jax.ShapeDtypeStruct((n, 128), jnp.float32),
                   jax.ShapeDtypeStruct((n, 128), jnp.float32),
                   jax.ShapeDtypeStruct((n, h_dim), jnp.float32)],
    )(s1lo, s1hi, cnt16, z1, b1.reshape(1, -1), W2l, W2r)

    seg2 = _make_seg_sum(n, n_streams, with_cnt=False)
    s2lo, s2hi = seg2(y2lo, y2hi, src_p, dst_p, zfeat, zcnt, ones_h)

    out = pl.pallas_call(
        _tc3_body,
        grid=grid,
        in_specs=[_row_spec(nb, 128), _row_spec(nb, 128), _row_spec(nb, 16),
                  _row_spec(nb, h_dim), _full_spec((1, h_dim))],
        out_specs=_row_spec(nb, h_dim),
        out_shape=jax.ShapeDtypeStruct((n, h_dim), jnp.float32),
    )(s2lo, s2hi, cnt16, z2, b2.reshape(1, -1))

    return out


# speedup vs baseline: 8.4131x; 1.2742x over previous
import functools

import jax
import jax.numpy as jnp
from jax import lax
from jax.experimental import pallas as pl
from jax.experimental.pallas import tpu as pltpu
from jax.experimental.pallas import tpu_sc as plsc

N_CORES = 2
N_SUBCORES = 16
STREAM = 128


def _make_seg_sum(n_nodes, stream, spt, with_cnt):
    own = -(-(n_nodes + 8) // (N_SUBCORES * 128)) * 128
    n_rows_acc = own * N_SUBCORES
    n_chunks = own // stream
    assert own % stream == 0
    last0 = (N_SUBCORES - 1) * own
    full_last = (n_nodes - last0) // stream
    tail = n_nodes - last0 - full_last * stream

    out_type = [
        jax.ShapeDtypeStruct((n_nodes, 128), jnp.float32),
        jax.ShapeDtypeStruct((n_nodes, 128), jnp.float32),
    ]
    scratch = [
        pltpu.VMEM((stream,), jnp.int32),
        pltpu.VMEM((stream,), jnp.int32),
        pltpu.VMEM((stream,), jnp.int32),
        pltpu.VMEM((stream,), jnp.int32),
        pltpu.VMEM((stream, 128), jnp.float32),
        pltpu.VMEM((stream, 128), jnp.float32),
        pltpu.VMEM_SHARED((n_rows_acc, 128), jnp.float32),
        pltpu.SemaphoreType.DMA,
        pltpu.SemaphoreType.DMA,
    ]
    if with_cnt:
        out_type.append(jax.ShapeDtypeStruct((n_nodes, 16), jnp.float32))
        scratch.append(pltpu.VMEM((128, 16), jnp.float32))
        scratch.append(pltpu.VMEM_SHARED((n_rows_acc, 16), jnp.float32))

    mesh = plsc.VectorSubcoreMesh(core_axis_name="c", subcore_axis_name="s")

    def body(ylo, yhi, srcp, dstp, zfeat, zcnt, ones_h, *rest):
        if with_cnt:
            (out_lo, out_hi, out_cnt, isrc0, isrc1, idst0, idst1,
             rows0, rows1, acc, gsem0, gsem1, onesv, cacc) = rest
        else:
            (out_lo, out_hi, isrc0, isrc1, idst0, idst1,
             rows0, rows1, acc, gsem0, gsem1) = rest
        isrc = [isrc0, isrc1]
        idst = [idst0, idst1]
        rows = [rows0, rows1]
        gsem = [gsem0, gsem1]
        tabs = [ylo, yhi]
        c = lax.axis_index("c")
        s = lax.axis_index("s")
        r0 = s * own

        pltpu.sync_copy(zfeat.at[pl.ds(0, stream)], rows0)
        for m in range(n_chunks):
            pltpu.sync_copy(rows0, acc.at[pl.ds(r0 + m * stream, stream)])

        if with_cnt:
            @pl.when(c == 1)
            def _():
                pltpu.sync_copy(zcnt, onesv)
                for m in range(own // 128):
                    pltpu.sync_copy(onesv,
                                    cacc.at[pl.ds(r0 + m * 128, 128)])
                pltpu.sync_copy(ones_h, onesv)

        plsc.subcore_barrier()

        g0 = s * spt

        def start_gather(g, b):
            off = (g0 + g) * stream
            pltpu.sync_copy(srcp.at[pl.ds(off, stream)], isrc[b])
            pltpu.sync_copy(dstp.at[pl.ds(off, stream)], idst[b])
            for cc in range(N_CORES):
                @pl.when(c == cc)
                def _():
                    pltpu.async_copy(tabs[cc].at[isrc[b]], rows[b], gsem[b])

        def wait_gather(b):
            for cc in range(N_CORES):
                @pl.when(c == cc)
                def _():
                    pltpu.make_async_copy(tabs[cc].at[isrc[b]], rows[b],
                                          gsem[b]).wait()

        start_gather(0, 0)
        start_gather(1, 1)

        def iter_body(k, carry):
            for b in range(2):
                g = k * 2 + b
                wait_gather(b)
                pltpu.sync_copy(rows[b], acc.at[idst[b]], add=True)
                if with_cnt:
                    @pl.when(c == 1)
                    def _():
                        pltpu.sync_copy(onesv.at[pl.ds(0, stream)],
                                        cacc.at[idst[b]], add=True)

                @pl.when(g + 2 < spt)
                def _():
                    start_gather(g + 2, b)
            return carry

        lax.fori_loop(0, spt // 2, iter_body, 0)

        plsc.subcore_barrier()

        out_feat = [out_lo, out_hi]

        def wb_feat(base, nrows):
            rst = rows0.at[pl.ds(0, nrows)]
            pltpu.sync_copy(acc.at[pl.ds(base, nrows)], rst)
            for cc in range(N_CORES):
                @pl.when(c == cc)
                def _():
                    pltpu.sync_copy(rst, out_feat[cc].at[pl.ds(base, nrows)])

        def wb_cnt(base, nrows):
            cst = onesv.at[pl.ds(0, nrows)]

            @pl.when(c == 1)
            def _():
                pltpu.sync_copy(cacc.at[pl.ds(base, nrows)], cst)
                pltpu.sync_copy(cst, out_cnt.at[pl.ds(base, nrows)])

        @pl.when(s < N_SUBCORES - 1)
        def _():
            for m in range(n_chunks):
                wb_feat(r0 + m * stream, stream)
                if with_cnt:
                    wb_cnt(r0 + m * stream, stream)

        @pl.when(s == N_SUBCORES - 1)
        def _():
            for m in range(full_last):
                wb_feat(last0 + m * stream, stream)
                if with_cnt:
                    wb_cnt(last0 + m * stream, stream)
            if tail:
                wb_feat(last0 + full_last * stream, tail)
                if with_cnt:
                    wb_cnt(last0 + full_last * stream, tail)

    return pl.kernel(
        body, out_type=out_type, mesh=mesh, scratch_types=scratch,
        compiler_params=pltpu.CompilerParams(use_tc_tiling_on_sc=False),
        name="seg_sum_sc")


def _tc1_body(x_ref, wl_ref, wr_ref, ylo_ref, yhi_ref, z_ref):
    xb = x_ref[...]
    yl = jnp.dot(xb, wl_ref[...], preferred_element_type=jnp.float32)
    ylo_ref[...] = yl[:, :128]
    yhi_ref[...] = yl[:, 128:]
    z_ref[...] = jnp.dot(xb, wr_ref[...], preferred_element_type=jnp.float32)


def _tc2_body(slo_ref, shi_ref, cnt_ref, z1_ref, b1_ref, w2l_ref, w2r_ref,
              ylo_ref, yhi_ref, z2_ref):
    inv = 1.0 / jnp.maximum(cnt_ref[...][:, 0:1], 1.0)
    sfull = jnp.concatenate([slo_ref[...], shi_ref[...]], axis=1)
    h = jnp.maximum(sfull * inv + b1_ref[...] + z1_ref[...], 0.0)
    y2 = jnp.dot(h, w2l_ref[...], preferred_element_type=jnp.float32)
    ylo_ref[...] = y2[:, :128]
    yhi_ref[...] = y2[:, 128:]
    z2_ref[...] = jnp.dot(h, w2r_ref[...], preferred_element_type=jnp.float32)


def _tc3_body(slo_ref, shi_ref, cnt_ref, z2_ref, b2_ref, out_ref):
    inv = 1.0 / jnp.maximum(cnt_ref[...][:, 0:1], 1.0)
    sfull = jnp.concatenate([slo_ref[...], shi_ref[...]], axis=1)
    out_ref[...] = sfull * inv + b2_ref[...] + z2_ref[...]


def _row_spec(nb, w):
    return pl.BlockSpec((nb, w), lambda i: (i, 0))


def _full_spec(shape):
    return pl.BlockSpec(shape, lambda i: tuple(0 for _ in shape))


def kernel(x, edge_index, W1l, b1, W1r, W2l, b2, W2r):
    n, d_in = x.shape
    h_dim = W1l.shape[1]
    e = edge_index.shape[1]

    stream1, stream2 = 80, 128

    def _spt(st):
        q = -(-e // (st * N_SUBCORES))
        return q + (q % 2)

    spt1, spt2 = _spt(stream1), _spt(stream2)
    e_pad = max(spt1 * stream1, spt2 * stream2) * N_SUBCORES
    pad = e_pad - e
    src_p = jnp.concatenate([edge_index[0], jnp.zeros((pad,), jnp.int32)])
    dst_p = jnp.concatenate([edge_index[1], jnp.full((pad,), n, jnp.int32)])

    zfeat = jnp.zeros((STREAM, 128), jnp.float32)
    zcnt = jnp.zeros((STREAM, 16), jnp.float32)
    ones_h = jnp.ones((STREAM, 16), jnp.float32)

    nb = 1000
    grid = (n // nb,)

    y1lo, y1hi, z1 = pl.pallas_call(
        _tc1_body,
        grid=grid,
        in_specs=[_row_spec(nb, d_in), _full_spec((d_in, h_dim)),
                  _full_spec((d_in, h_dim))],
        out_specs=[_row_spec(nb, 128), _row_spec(nb, 128),
                   _row_spec(nb, h_dim)],
        out_shape=[jax.ShapeDtypeStruct((n, 128), jnp.float32),
                   jax.ShapeDtypeStruct((n, 128), jnp.float32),
                   jax.ShapeDtypeStruct((n, h_dim), jnp.float32)],
    )(x, W1l, W1r)

    seg1 = _make_seg_sum(n, stream1, spt1, with_cnt=True)
    s1lo, s1hi, cnt16 = seg1(y1lo, y1hi, src_p, dst_p, zfeat, zcnt, ones_h)

    y2lo, y2hi, z2 = pl.pallas_call(
        _tc2_body,
        grid=grid,
        in_specs=[_row_spec(nb, 128), _row_spec(nb, 128), _row_spec(nb, 16),
                  _row_spec(nb, h_dim), _full_spec((1, h_dim)),
                  _full_spec((h_dim, h_dim)), _full_spec((h_dim, h_dim))],
        out_specs=[_row_spec(nb, 128), _row_spec(nb, 128),
                   _row_spec(nb, h_dim)],
        out_shape=[jax.ShapeDtypeStruct((n, 128), jnp.float32),
                   jax.ShapeDtypeStruct((n, 128), jnp.float32),
                   jax.ShapeDtypeStruct((n, h_dim), jnp.float32)],
    )(s1lo, s1hi, cnt16, z1, b1.reshape(1, -1), W2l, W2r)

    seg2 = _make_seg_sum(n, stream2, spt2, with_cnt=False)
    s2lo, s2hi = seg2(y2lo, y2hi, src_p, dst_p, zfeat, zcnt, ones_h)

    out = pl.pallas_call(
        _tc3_body,
        grid=grid,
        in_specs=[_row_spec(nb, 128), _row_spec(nb, 128), _row_spec(nb, 16),
                  _row_spec(nb, h_dim), _full_spec((1, h_dim))],
        out_specs=_row_spec(nb, h_dim),
        out_shape=jax.ShapeDtypeStruct((n, h_dim), jnp.float32),
    )(s2lo, s2hi, cnt16, z2, b2.reshape(1, -1))

    return out
